# baseline (speedup 1.0000x reference)
import functools

import jax
import jax.numpy as jnp
from jax import lax
from jax.experimental import pallas as pl
from jax.experimental.pallas import tpu as pltpu
from jax.experimental.pallas import tpu_sc as plsc

_N = 10000
_E = 320000
_D = 128
_DE = 4

_NC = 2
_NS = 16
_L = 16
_NW = _NC * _NS
_EPW = _E // _NW
_C = 40
_NCHUNK = _EPW // _C
_RPT = 632
_NP = _NS * _RPT


def _pre_body(x_ref, wf_ref, wt_ref, b_ref, xf_ref, xt_ref):
    x = x_ref[...]
    xf_ref[...] = (
        jnp.dot(x, wf_ref[...], preferred_element_type=jnp.float32) + b_ref[...]
    )
    xt_ref[...] = jnp.dot(x, wt_ref[...], preferred_element_type=jnp.float32)


def _pre(x, wf, wt, b2d):
    return pl.pallas_call(
        _pre_body,
        out_shape=[jax.ShapeDtypeStruct((_N, _D), jnp.float32)] * 2,
    )(x, wf, wt, b2d)


_MESH = plsc.VectorSubcoreMesh(core_axis_name="c", subcore_axis_name="s")


@functools.partial(
    pl.kernel,
    out_type=jax.ShapeDtypeStruct((_NC, _N, _D), jnp.float32),
    mesh=_MESH,
    scratch_types=[
        pltpu.VMEM((2 * _C,), jnp.int32),
        pltpu.VMEM((2 * _C,), jnp.int32),
        pltpu.VMEM((_C,), jnp.int32),
        pltpu.VMEM((_C,), jnp.int32),
        pltpu.VMEM((2 * _C * _DE,), jnp.float32),
        pltpu.VMEM((2, _C, _D), jnp.float32),
        pltpu.VMEM((2, _C, _D), jnp.float32),
        pltpu.VMEM((2, _C, _D), jnp.float32),
        pltpu.VMEM((_DE, _D), jnp.float32),
        pltpu.VMEM_SHARED((_NP, _D), jnp.float32),
        pltpu.SemaphoreType.DMA,
        pltpu.SemaphoreType.DMA,
        pltpu.SemaphoreType.DMA,
        pltpu.SemaphoreType.DMA,
        pltpu.SemaphoreType.DMA,
        pltpu.SemaphoreType.DMA,
        pltpu.SemaphoreType.DMA,
        pltpu.SemaphoreType.DMA,
        pltpu.SemaphoreType.DMA,
        pltpu.SemaphoreType.DMA,
    ],
    compiler_params=pltpu.CompilerParams(needs_layout_passes=False),
)
def _sc_agg(xf_hbm, xt_hbm, fidx_hbm, tidx_hbm, ef_hbm, we_hbm, zeros_hbm,
            out_hbm, fidx_v, tidx_v, sidx0_v, sidx1_v, ef_v, xf_v, xt_v,
            msg_v, we_v, agg_sh, gf0, gt0, gf1, gt1, ss0, ss1, it0, it1,
            es0, es1):
    c = lax.axis_index("c")
    s = lax.axis_index("s")
    wid = s * _NC + c
    gf = (gf0, gf1)
    gt = (gt0, gt1)
    ss = (ss0, ss1)
    it = (it0, it1)
    es = (es0, es1)
    sidx = (sidx0_v, sidx1_v)

    pltpu.sync_copy(we_hbm, we_v)

    base_row = s * _RPT
    pltpu.sync_copy(
        zeros_hbm.at[pl.ds(base_row, _RPT)],
        agg_sh.at[pl.ds(base_row, _RPT)],
    )
    plsc.subcore_barrier()

    wsl = [[we_v[k, pl.ds(j * _L, _L)] for j in range(_D // _L)]
           for k in range(_DE)]
    ebase = wid * _EPW

    def _ft_copies(i, par):
        a = pltpu.make_async_copy(fidx_hbm.at[pl.ds(ebase + i * _C, _C)],
                                  fidx_v.at[pl.ds(par * _C, _C)], it[par])
        b = pltpu.make_async_copy(tidx_hbm.at[pl.ds(ebase + i * _C, _C)],
                                  tidx_v.at[pl.ds(par * _C, _C)], it[par])
        return a, b

    def _ef_copy(i, par):
        return pltpu.make_async_copy(
            ef_hbm.at[pl.ds((ebase + i * _C) * _DE, _C * _DE)],
            ef_v.at[pl.ds(par * _C * _DE, _C * _DE)], es[par])

    def issue_idxft(i, par):
        a, b = _ft_copies(i, par)
        a.start()
        b.start()

    def wait_idxft(i, par):
        a, b = _ft_copies(i, par)
        a.wait()
        b.wait()

    def issue_ef(i, par):
        _ef_copy(i, par).start()

    def wait_ef(i, par):
        _ef_copy(i, par).wait()

    def _g_copies(par):
        fi = fidx_v.at[pl.ds(par * _C, _C)]
        ti = tidx_v.at[pl.ds(par * _C, _C)]
        a = pltpu.make_async_copy(xf_hbm.at[fi], xf_v.at[par], gf[par])
        b = pltpu.make_async_copy(xt_hbm.at[ti], xt_v.at[par], gt[par])
        return a, b

    def issue_gather(par):
        a, b = _g_copies(par)
        a.start()
        b.start()

    def wait_gather(par):
        a, b = _g_copies(par)
        a.wait()
        b.wait()

    iota4 = lax.rem(lax.iota(jnp.int32, _L), _DE)
    ksplat = [jnp.full((_L,), k, jnp.int32) for k in range(_DE)]
    _dnums = lax.GatherDimensionNumbers(
        offset_dims=(), collapsed_slice_dims=(0,), start_index_map=(0,))

    def _bcast_lane(vec, ks):
        return lax.gather(
            vec, ks[:, None], dimension_numbers=_dnums, slice_sizes=(1,),
            mode=lax.GatherScatterMode.PROMISE_IN_BOUNDS)

    def compute(par):
        pbase = jnp.full((_L,), par * _C * _DE, jnp.int32) + iota4

        @plsc.parallel_loop(0, _C, step=1, unroll=16)
        def edge(e):
            quad = plsc.load_gather(ef_v, [pbase + e * _DE])
            ev = [_bcast_lane(quad, ksplat[k]) for k in range(_DE)]
            for j in range(_D // _L):
                sl = pl.ds(j * _L, _L)
                m = xf_v[par, e, sl] + xt_v[par, e, sl]
                for k in range(_DE):
                    m = m + ev[k] * wsl[k][j]
                msg_v[par, e, sl] = jnp.maximum(m, 0.0)

    _snap_starts = sorted({min(q * _L, _C - _L)
                           for q in range((_C + _L - 1) // _L)})

    def snap_sidx(par):
        for q0 in _snap_starts:
            sidx[par][pl.ds(q0, _L)] = tidx_v[pl.ds(par * _C + q0, _L)]

    def issue_scatter(par):
        pltpu.async_copy(msg_v.at[par], agg_sh.at[sidx[par]], ss[par],
                         add=True)

    def wait_scatter(par):
        pltpu.make_async_copy(msg_v.at[par], agg_sh.at[sidx[par]],
                              ss[par]).wait()

    def load_idx_sync(i, par):
        pltpu.sync_copy(fidx_hbm.at[pl.ds(ebase + i * _C, _C)],
                        fidx_v.at[pl.ds(par * _C, _C)])
        pltpu.sync_copy(tidx_hbm.at[pl.ds(ebase + i * _C, _C)],
                        tidx_v.at[pl.ds(par * _C, _C)])
        pltpu.sync_copy(ef_hbm.at[pl.ds((ebase + i * _C) * _DE, _C * _DE)],
                        ef_v.at[pl.ds(par * _C * _DE, _C * _DE)])

    load_idx_sync(0, 0)
    issue_gather(0)
    load_idx_sync(1, 1)
    issue_gather(1)
    wait_gather(0)
    snap_sidx(0)
    issue_idxft(2, 0)
    compute(0)
    issue_scatter(0)
    issue_ef(2, 0)

    _K = (_NCHUNK - 2) // 2

    def body(k, carry):
        i1 = 2 * k + 1
        i2 = 2 * k + 2
        wait_idxft(i2, 0)
        issue_gather(0)
        wait_gather(1)

        @pl.when(k > 0)
        def _():
            wait_scatter(1)

        snap_sidx(1)
        issue_idxft(i1 + 2, 1)

        @pl.when(k > 0)
        def _():
            wait_ef(i1, 1)

        compute(1)
        issue_scatter(1)
        issue_ef(i1 + 2, 1)
        wait_idxft(i1 + 2, 1)
        issue_gather(1)

        wait_gather(0)
        wait_scatter(0)
        snap_sidx(0)

        @pl.when(k < _K - 1)
        def _():
            issue_idxft(i2 + 2, 0)

        wait_ef(i2, 0)
        compute(0)
        issue_scatter(0)

        @pl.when(k < _K - 1)
        def _():
            issue_ef(i2 + 2, 0)

        return carry

    lax.fori_loop(0, _K, body, 0)

    last = _NCHUNK - 1
    wait_gather(1)
    wait_scatter(1)
    snap_sidx(1)
    wait_ef(last, 1)
    compute(1)
    issue_scatter(1)

    wait_scatter(0)
    wait_scatter(1)
    plsc.subcore_barrier()

    @pl.when(s < _NS - 1)
    def _():
        pltpu.sync_copy(
            agg_sh.at[pl.ds(base_row, _RPT)],
            out_hbm.at[c, pl.ds(base_row, _RPT)],
        )

    @pl.when(s == _NS - 1)
    def _():
        pltpu.sync_copy(
            agg_sh.at[pl.ds((_NS - 1) * _RPT, _N - (_NS - 1) * _RPT)],
            out_hbm.at[c, pl.ds((_NS - 1) * _RPT, _N - (_NS - 1) * _RPT)],
        )


def _post_body(x_ref, a0_ref, a1_ref, w_ref, b_ref, o_ref):
    agg = a0_ref[...] + a1_ref[...]
    h = jnp.dot(agg, w_ref[...], preferred_element_type=jnp.float32) + b_ref[...]
    o_ref[...] = x_ref[...] + jnp.maximum(h, 0.0)


def _post(x, a0, a1, w, b2d):
    return pl.pallas_call(
        _post_body,
        out_shape=jax.ShapeDtypeStruct((_N, _D), jnp.float32),
    )(x, a0, a1, w, b2d)


def kernel(node_features, edge_features, from_idx, to_idx, W_msg, b_msg,
           W_mlp, b_mlp):
    wf = W_msg[:_D]
    wt = W_msg[_D:2 * _D]
    we = W_msg[2 * _D:]
    xf, xt = _pre(node_features, wf, wt, b_msg.reshape(1, _D))
    zeros = jnp.zeros((_NP, _D), jnp.float32)
    parts = _sc_agg(xf, xt, from_idx, to_idx, edge_features.reshape(-1), we,
                    zeros)
    return _post(node_features, parts[0], parts[1], W_mlp,
                 b_mlp.reshape(1, _D))


# speedup vs baseline: 1.1021x; 1.1021x over previous
import functools

import jax
import jax.numpy as jnp
from jax import lax
from jax.experimental import pallas as pl
from jax.experimental.pallas import tpu as pltpu
from jax.experimental.pallas import tpu_sc as plsc

_N = 10000
_E = 320000
_D = 128
_DE = 4

_NC = 2
_NS = 16
_L = 16
_NW = _NC * _NS
_EPW = _E // _NW
_C = 40
_NCHUNK = _EPW // _C
_RPT = 632
_NP = _NS * _RPT


def _pre_body(x_ref, wf_ref, wt_ref, b_ref, xf_ref, xt_ref):
    x = x_ref[...]
    xf_ref[...] = (
        jnp.dot(x, wf_ref[...], preferred_element_type=jnp.float32) + b_ref[...]
    )
    xt_ref[...] = jnp.dot(x, wt_ref[...], preferred_element_type=jnp.float32)


def _pre(x, wf, wt, b2d):
    return pl.pallas_call(
        _pre_body,
        out_shape=[jax.ShapeDtypeStruct((_N, _D), jnp.float32)] * 2,
    )(x, wf, wt, b2d)


_MESH = plsc.VectorSubcoreMesh(core_axis_name="c", subcore_axis_name="s")


@functools.partial(
    pl.kernel,
    out_type=jax.ShapeDtypeStruct((_NC, _N, _D), jnp.float32),
    mesh=_MESH,
    scratch_types=[
        pltpu.VMEM((2 * _C,), jnp.int32),
        pltpu.VMEM((2 * _C,), jnp.int32),
        pltpu.VMEM((_C,), jnp.int32),
        pltpu.VMEM((_C,), jnp.int32),
        pltpu.VMEM((2 * _C * _DE,), jnp.float32),
        pltpu.VMEM((2, _C, _D), jnp.float32),
        pltpu.VMEM((2, _C, _D), jnp.float32),
        pltpu.VMEM((2, _C, _D), jnp.float32),
        pltpu.VMEM((_DE, _D), jnp.float32),
        pltpu.VMEM_SHARED((_NP, _D), jnp.float32),
        pltpu.SemaphoreType.DMA,
        pltpu.SemaphoreType.DMA,
        pltpu.SemaphoreType.DMA,
        pltpu.SemaphoreType.DMA,
        pltpu.SemaphoreType.DMA,
        pltpu.SemaphoreType.DMA,
        pltpu.SemaphoreType.DMA,
        pltpu.SemaphoreType.DMA,
        pltpu.SemaphoreType.DMA,
        pltpu.SemaphoreType.DMA,
    ],
    compiler_params=pltpu.CompilerParams(needs_layout_passes=False),
)
def _sc_agg(xf_hbm, xt_hbm, fidx_hbm, tidx_hbm, ef_hbm, we_hbm, zeros_hbm,
            out_hbm, fidx_v, tidx_v, sidx0_v, sidx1_v, ef_v, xf_v, xt_v,
            msg_v, we_v, agg_sh, gf0, gt0, gf1, gt1, ss0, ss1, it0, it1,
            es0, es1):
    c = lax.axis_index("c")
    s = lax.axis_index("s")
    wid = s * _NC + c
    gf = (gf0, gf1)
    gt = (gt0, gt1)
    ss = (ss0, ss1)
    it = (it0, it1)
    es = (es0, es1)
    sidx = (sidx0_v, sidx1_v)

    pltpu.sync_copy(we_hbm, we_v)

    base_row = s * _RPT
    pltpu.sync_copy(
        zeros_hbm.at[pl.ds(base_row, _RPT)],
        agg_sh.at[pl.ds(base_row, _RPT)],
    )
    plsc.subcore_barrier()

    wsl = [[we_v[k, pl.ds(j * _L, _L)] for j in range(_D // _L)]
           for k in range(_DE)]
    ebase = wid * _EPW

    def _ft_copies(i, par):
        a = pltpu.make_async_copy(fidx_hbm.at[pl.ds(ebase + i * _C, _C)],
                                  fidx_v.at[pl.ds(par * _C, _C)], it[par])
        b = pltpu.make_async_copy(tidx_hbm.at[pl.ds(ebase + i * _C, _C)],
                                  tidx_v.at[pl.ds(par * _C, _C)], it[par])
        return a, b

    def _ef_copy(i, par):
        return pltpu.make_async_copy(
            ef_hbm.at[pl.ds((ebase + i * _C) * _DE, _C * _DE)],
            ef_v.at[pl.ds(par * _C * _DE, _C * _DE)], es[par])

    def issue_idxft(i, par):
        a, b = _ft_copies(i, par)
        a.start()
        b.start()

    def wait_idxft(i, par):
        a, b = _ft_copies(i, par)
        a.wait()
        b.wait()

    def issue_ef(i, par):
        _ef_copy(i, par).start()

    def wait_ef(i, par):
        _ef_copy(i, par).wait()

    def _g_copies(par):
        fi = fidx_v.at[pl.ds(par * _C, _C)]
        ti = tidx_v.at[pl.ds(par * _C, _C)]
        a = pltpu.make_async_copy(xf_hbm.at[fi], xf_v.at[par], gf[par])
        b = pltpu.make_async_copy(xt_hbm.at[ti], xt_v.at[par], gt[par])
        return a, b

    def issue_gather(par):
        a, b = _g_copies(par)
        a.start()
        b.start()

    def wait_gather(par):
        a, b = _g_copies(par)
        a.wait()
        b.wait()

    iota4 = lax.rem(lax.iota(jnp.int32, _L), _DE)
    ksplat = [jnp.full((_L,), k, jnp.int32) for k in range(_DE)]
    _dnums = lax.GatherDimensionNumbers(
        offset_dims=(), collapsed_slice_dims=(0,), start_index_map=(0,))

    def _bcast_lane(vec, ks):
        return lax.gather(
            vec, ks[:, None], dimension_numbers=_dnums, slice_sizes=(1,),
            mode=lax.GatherScatterMode.PROMISE_IN_BOUNDS)

    def compute(par):
        pbase = jnp.full((_L,), par * _C * _DE, jnp.int32) + iota4

        @plsc.parallel_loop(0, _C, step=1, unroll=8)
        def edge(e):
            quad = plsc.load_gather(ef_v, [pbase + e * _DE])
            ev = [_bcast_lane(quad, ksplat[k]) for k in range(_DE)]
            for j in range(_D // _L):
                sl = pl.ds(j * _L, _L)
                m = xf_v[par, e, sl] + xt_v[par, e, sl]
                for k in range(_DE):
                    m = m + ev[k] * wsl[k][j]
                msg_v[par, e, sl] = jnp.maximum(m, 0.0)

    _snap_starts = sorted({min(q * _L, _C - _L)
                           for q in range((_C + _L - 1) // _L)})

    def snap_sidx(par):
        for q0 in _snap_starts:
            sidx[par][pl.ds(q0, _L)] = tidx_v[pl.ds(par * _C + q0, _L)]

    def issue_scatter(par):
        pltpu.async_copy(msg_v.at[par], agg_sh.at[sidx[par]], ss[par],
                         add=True)

    def wait_scatter(par):
        pltpu.make_async_copy(msg_v.at[par], agg_sh.at[sidx[par]],
                              ss[par]).wait()

    def load_idx_sync(i, par):
        pltpu.sync_copy(fidx_hbm.at[pl.ds(ebase + i * _C, _C)],
                        fidx_v.at[pl.ds(par * _C, _C)])
        pltpu.sync_copy(tidx_hbm.at[pl.ds(ebase + i * _C, _C)],
                        tidx_v.at[pl.ds(par * _C, _C)])
        pltpu.sync_copy(ef_hbm.at[pl.ds((ebase + i * _C) * _DE, _C * _DE)],
                        ef_v.at[pl.ds(par * _C * _DE, _C * _DE)])

    load_idx_sync(0, 0)
    issue_gather(0)
    load_idx_sync(1, 1)
    issue_gather(1)
    wait_gather(0)
    snap_sidx(0)
    issue_idxft(2, 0)
    compute(0)
    issue_scatter(0)
    issue_ef(2, 0)

    _K = (_NCHUNK - 2) // 2

    def body(k, carry):
        i1 = 2 * k + 1
        i2 = 2 * k + 2
        wait_idxft(i2, 0)
        issue_gather(0)
        wait_gather(1)

        @pl.when(k > 0)
        def _():
            wait_scatter(1)

        snap_sidx(1)
        issue_idxft(i1 + 2, 1)

        @pl.when(k > 0)
        def _():
            wait_ef(i1, 1)

        compute(1)
        issue_scatter(1)
        issue_ef(i1 + 2, 1)
        wait_idxft(i1 + 2, 1)
        issue_gather(1)

        wait_gather(0)
        wait_scatter(0)
        snap_sidx(0)

        @pl.when(k < _K - 1)
        def _():
            issue_idxft(i2 + 2, 0)

        wait_ef(i2, 0)
        compute(0)
        issue_scatter(0)

        @pl.when(k < _K - 1)
        def _():
            issue_ef(i2 + 2, 0)

        return carry

    lax.fori_loop(0, _K, body, 0)

    last = _NCHUNK - 1
    wait_gather(1)
    wait_scatter(1)
    snap_sidx(1)
    wait_ef(last, 1)
    compute(1)
    issue_scatter(1)

    wait_scatter(0)
    wait_scatter(1)
    plsc.subcore_barrier()

    @pl.when(s < _NS - 1)
    def _():
        pltpu.sync_copy(
            agg_sh.at[pl.ds(base_row, _RPT)],
            out_hbm.at[c, pl.ds(base_row, _RPT)],
        )

    @pl.when(s == _NS - 1)
    def _():
        pltpu.sync_copy(
            agg_sh.at[pl.ds((_NS - 1) * _RPT, _N - (_NS - 1) * _RPT)],
            out_hbm.at[c, pl.ds((_NS - 1) * _RPT, _N - (_NS - 1) * _RPT)],
        )


def _post_body(x_ref, a0_ref, a1_ref, w_ref, b_ref, o_ref):
    agg = a0_ref[...] + a1_ref[...]
    h = jnp.dot(agg, w_ref[...], preferred_element_type=jnp.float32) + b_ref[...]
    o_ref[...] = x_ref[...] + jnp.maximum(h, 0.0)


def _post(x, a0, a1, w, b2d):
    return pl.pallas_call(
        _post_body,
        out_shape=jax.ShapeDtypeStruct((_N, _D), jnp.float32),
    )(x, a0, a1, w, b2d)


def kernel(node_features, edge_features, from_idx, to_idx, W_msg, b_msg,
           W_mlp, b_mlp):
    wf = W_msg[:_D]
    wt = W_msg[_D:2 * _D]
    we = W_msg[2 * _D:]
    xf, xt = _pre(node_features, wf, wt, b_msg.reshape(1, _D))
    zeros = jnp.zeros((_NP, _D), jnp.float32)
    parts = _sc_agg(xf, xt, from_idx, to_idx, edge_features.reshape(-1), we,
                    zeros)
    return _post(node_features, parts[0], parts[1], W_mlp,
                 b_mlp.reshape(1, _D))


# speedup vs baseline: 1.2104x; 1.0982x over previous
import functools

import jax
import jax.numpy as jnp
from jax import lax
from jax.experimental import pallas as pl
from jax.experimental.pallas import tpu as pltpu
from jax.experimental.pallas import tpu_sc as plsc

_N = 10000
_E = 320000
_D = 128
_DE = 4

_NC = 2
_NS = 16
_L = 16
_NW = _NC * _NS
_EPW = _E // _NW
_C = 40
_NCHUNK = _EPW // _C
_RPT = 632
_NP = _NS * _RPT


def _pre_body(x_ref, wf_ref, wt_ref, b_ref, xf_ref, xt_ref):
    x = x_ref[...]
    xf_ref[...] = (
        jnp.dot(x, wf_ref[...], preferred_element_type=jnp.float32) + b_ref[...]
    )
    xt_ref[...] = jnp.dot(x, wt_ref[...], preferred_element_type=jnp.float32)


def _pre(x, wf, wt, b2d):
    return pl.pallas_call(
        _pre_body,
        out_shape=[jax.ShapeDtypeStruct((_N, _D), jnp.float32)] * 2,
    )(x, wf, wt, b2d)


_MESH = plsc.VectorSubcoreMesh(core_axis_name="c", subcore_axis_name="s")


@functools.partial(
    pl.kernel,
    out_type=jax.ShapeDtypeStruct((_NC, _N, _D), jnp.float32),
    mesh=_MESH,
    scratch_types=[
        pltpu.VMEM((2 * _C,), jnp.int32),
        pltpu.VMEM((2 * _C,), jnp.int32),
        pltpu.VMEM((_C,), jnp.int32),
        pltpu.VMEM((_C,), jnp.int32),
        pltpu.VMEM((2 * _C * _DE,), jnp.float32),
        pltpu.VMEM((2, _C, _D), jnp.float32),
        pltpu.VMEM((2, _C, _D), jnp.float32),
        pltpu.VMEM((2, _C, _D), jnp.float32),
        pltpu.VMEM((_DE, _D), jnp.float32),
        pltpu.VMEM_SHARED((_NP, _D), jnp.float32),
        pltpu.SemaphoreType.DMA,
        pltpu.SemaphoreType.DMA,
        pltpu.SemaphoreType.DMA,
        pltpu.SemaphoreType.DMA,
        pltpu.SemaphoreType.DMA,
        pltpu.SemaphoreType.DMA,
        pltpu.SemaphoreType.DMA,
        pltpu.SemaphoreType.DMA,
        pltpu.SemaphoreType.DMA,
        pltpu.SemaphoreType.DMA,
    ],
    compiler_params=pltpu.CompilerParams(needs_layout_passes=False),
)
def _sc_agg(xf_hbm, xt_hbm, fidx_hbm, tidx_hbm, ef_hbm, we_hbm, zeros_hbm,
            out_hbm, fidx_v, tidx_v, sidx0_v, sidx1_v, ef_v, xf_v, xt_v,
            msg_v, we_v, agg_sh, gf0, gt0, gf1, gt1, ss0, ss1, it0, it1,
            es0, es1):
    c = lax.axis_index("c")
    s = lax.axis_index("s")
    wid = s * _NC + c
    gf = (gf0, gf1)
    gt = (gt0, gt1)
    ss = (ss0, ss1)
    it = (it0, it1)
    es = (es0, es1)
    sidx = (sidx0_v, sidx1_v)

    pltpu.sync_copy(we_hbm, we_v)

    base_row = s * _RPT
    pltpu.sync_copy(
        zeros_hbm.at[pl.ds(base_row, _RPT)],
        agg_sh.at[pl.ds(base_row, _RPT)],
    )
    plsc.subcore_barrier()

    wsl = [[we_v[k, pl.ds(j * _L, _L)] for j in range(_D // _L)]
           for k in range(_DE)]
    ebase = wid * _EPW

    def _ft_copies(i, par):
        a = pltpu.make_async_copy(fidx_hbm.at[pl.ds(ebase + i * _C, _C)],
                                  fidx_v.at[pl.ds(par * _C, _C)], it[par])
        b = pltpu.make_async_copy(tidx_hbm.at[pl.ds(ebase + i * _C, _C)],
                                  tidx_v.at[pl.ds(par * _C, _C)], it[par])
        return a, b

    def _ef_copy(i, par):
        return pltpu.make_async_copy(
            ef_hbm.at[pl.ds((ebase + i * _C) * _DE, _C * _DE)],
            ef_v.at[pl.ds(par * _C * _DE, _C * _DE)], es[par])

    def issue_idxft(i, par):
        a, b = _ft_copies(i, par)
        a.start()
        b.start()

    def wait_idxft(i, par):
        a, b = _ft_copies(i, par)
        a.wait()
        b.wait()

    def issue_ef(i, par):
        _ef_copy(i, par).start()

    def wait_ef(i, par):
        _ef_copy(i, par).wait()

    def _g_copies(par):
        fi = fidx_v.at[pl.ds(par * _C, _C)]
        ti = tidx_v.at[pl.ds(par * _C, _C)]
        a = pltpu.make_async_copy(xf_hbm.at[fi], xf_v.at[par], gf[par])
        b = pltpu.make_async_copy(xt_hbm.at[ti], xt_v.at[par], gt[par])
        return a, b

    def issue_gather(par):
        a, b = _g_copies(par)
        a.start()
        b.start()

    def wait_gather(par):
        a, b = _g_copies(par)
        a.wait()
        b.wait()

    iota4 = lax.rem(lax.iota(jnp.int32, _L), _DE)
    ksplat = [jnp.full((_L,), k, jnp.int32) for k in range(_DE)]
    _dnums = lax.GatherDimensionNumbers(
        offset_dims=(), collapsed_slice_dims=(0,), start_index_map=(0,))

    def _bcast_lane(vec, ks):
        return lax.gather(
            vec, ks[:, None], dimension_numbers=_dnums, slice_sizes=(1,),
            mode=lax.GatherScatterMode.PROMISE_IN_BOUNDS)

    def compute(par):
        pbase = jnp.full((_L,), par * _C * _DE, jnp.int32) + iota4

        @plsc.parallel_loop(0, _C, step=1, unroll=10)
        def edge(e):
            quad = plsc.load_gather(ef_v, [pbase + e * _DE])
            ev = [_bcast_lane(quad, ksplat[k]) for k in range(_DE)]
            for j in range(_D // _L):
                sl = pl.ds(j * _L, _L)
                m = xf_v[par, e, sl] + xt_v[par, e, sl]
                for k in range(_DE):
                    m = m + ev[k] * wsl[k][j]
                msg_v[par, e, sl] = jnp.maximum(m, 0.0)

    _snap_starts = sorted({min(q * _L, _C - _L)
                           for q in range((_C + _L - 1) // _L)})

    def snap_sidx(par):
        for q0 in _snap_starts:
            sidx[par][pl.ds(q0, _L)] = tidx_v[pl.ds(par * _C + q0, _L)]

    def issue_scatter(par):
        pltpu.async_copy(msg_v.at[par], agg_sh.at[sidx[par]], ss[par],
                         add=True)

    def wait_scatter(par):
        pltpu.make_async_copy(msg_v.at[par], agg_sh.at[sidx[par]],
                              ss[par]).wait()

    def load_idx_sync(i, par):
        pltpu.sync_copy(fidx_hbm.at[pl.ds(ebase + i * _C, _C)],
                        fidx_v.at[pl.ds(par * _C, _C)])
        pltpu.sync_copy(tidx_hbm.at[pl.ds(ebase + i * _C, _C)],
                        tidx_v.at[pl.ds(par * _C, _C)])
        pltpu.sync_copy(ef_hbm.at[pl.ds((ebase + i * _C) * _DE, _C * _DE)],
                        ef_v.at[pl.ds(par * _C * _DE, _C * _DE)])

    load_idx_sync(0, 0)
    issue_gather(0)
    load_idx_sync(1, 1)
    issue_gather(1)
    wait_gather(0)
    snap_sidx(0)
    issue_idxft(2, 0)
    compute(0)
    issue_scatter(0)
    issue_ef(2, 0)

    _K = (_NCHUNK - 2) // 2

    def body(k, carry):
        i1 = 2 * k + 1
        i2 = 2 * k + 2
        wait_idxft(i2, 0)
        issue_gather(0)
        wait_gather(1)

        @pl.when(k > 0)
        def _():
            wait_scatter(1)

        snap_sidx(1)
        issue_idxft(i1 + 2, 1)

        @pl.when(k > 0)
        def _():
            wait_ef(i1, 1)

        compute(1)
        issue_scatter(1)
        issue_ef(i1 + 2, 1)
        wait_idxft(i1 + 2, 1)
        issue_gather(1)

        wait_gather(0)
        wait_scatter(0)
        snap_sidx(0)

        @pl.when(k < _K - 1)
        def _():
            issue_idxft(i2 + 2, 0)

        wait_ef(i2, 0)
        compute(0)
        issue_scatter(0)

        @pl.when(k < _K - 1)
        def _():
            issue_ef(i2 + 2, 0)

        return carry

    lax.fori_loop(0, _K, body, 0)

    last = _NCHUNK - 1
    wait_gather(1)
    wait_scatter(1)
    snap_sidx(1)
    wait_ef(last, 1)
    compute(1)
    issue_scatter(1)

    wait_scatter(0)
    wait_scatter(1)
    plsc.subcore_barrier()

    @pl.when(s < _NS - 1)
    def _():
        pltpu.sync_copy(
            agg_sh.at[pl.ds(base_row, _RPT)],
            out_hbm.at[c, pl.ds(base_row, _RPT)],
        )

    @pl.when(s == _NS - 1)
    def _():
        pltpu.sync_copy(
            agg_sh.at[pl.ds((_NS - 1) * _RPT, _N - (_NS - 1) * _RPT)],
            out_hbm.at[c, pl.ds((_NS - 1) * _RPT, _N - (_NS - 1) * _RPT)],
        )


def _post_body(x_ref, a0_ref, a1_ref, w_ref, b_ref, o_ref):
    agg = a0_ref[...] + a1_ref[...]
    h = jnp.dot(agg, w_ref[...], preferred_element_type=jnp.float32) + b_ref[...]
    o_ref[...] = x_ref[...] + jnp.maximum(h, 0.0)


def _post(x, a0, a1, w, b2d):
    return pl.pallas_call(
        _post_body,
        out_shape=jax.ShapeDtypeStruct((_N, _D), jnp.float32),
    )(x, a0, a1, w, b2d)


def kernel(node_features, edge_features, from_idx, to_idx, W_msg, b_msg,
           W_mlp, b_mlp):
    wf = W_msg[:_D]
    wt = W_msg[_D:2 * _D]
    we = W_msg[2 * _D:]
    xf, xt = _pre(node_features, wf, wt, b_msg.reshape(1, _D))
    zeros = jnp.zeros((_NP, _D), jnp.float32)
    parts = _sc_agg(xf, xt, from_idx, to_idx, edge_features.reshape(-1), we,
                    zeros)
    return _post(node_features, parts[0], parts[1], W_mlp,
                 b_mlp.reshape(1, _D))
